# baseline (speedup 1.0000x reference)
import jax
import jax.numpy as jnp
from jax.experimental import pallas as pl
from jax.experimental.pallas import tpu as pltpu

_N = 8
_S = 4
_H = 8
_ETA = 1.0

_DN_T = (((1,), (1,)), ((), ()))
_DN_N = (((1,), (0,)), ((), ()))
_NK = _N * (_H + _S) + _N


def _glo_kernel(x_ref, w_ref, b_ref, sw_ref, mean_ref, slope_ref,
                center_ref, lu_ref, ld_ref, bias_ref, db_ref, out_ref,
                w_s, fw_s, gt_s, sm_s):
    f32 = jnp.float32
    bf16 = jnp.bfloat16

    @pl.when(pl.program_id(0) == 0)
    def _prep():
        w = w_ref[...]
        w_s[...] = w.astype(bf16)
        wcat = jnp.concatenate(
            [lu_ref[n] for n in range(_N)] +
            [sw_ref[n] for n in range(_N)], axis=1)
        fw = jax.lax.dot_general(w, wcat, (((0,), (0,)), ((), ())),
                                 preferred_element_type=f32)
        fm = jax.lax.dot_general(w, mean_ref[...], (((0,), (1,)), ((), ())),
                                 preferred_element_type=f32)
        fw_s[...] = jnp.concatenate([fw, fm], axis=1).astype(bf16)
        b = b_ref[...]
        bw = jax.lax.dot_general(b, wcat, _DN_N, preferred_element_type=f32)
        bm = jax.lax.dot_general(b, mean_ref[...], _DN_T,
                                 preferred_element_type=f32)
        sm_s[1:2, 0:_NK] = jnp.concatenate([bw, bm], axis=1)
        mean = mean_ref[...]
        mw_full = jax.lax.dot_general(
            mean, wcat[:, _N * _H:], _DN_N, preferred_element_type=f32)
        cols_s = jax.lax.broadcasted_iota(jnp.int32, (_N, _N * _S), 1) // _S
        rows_s = jax.lax.broadcasted_iota(jnp.int32, (_N, _N * _S), 0)
        sm_s[0:1, 0:_N * _S] = jnp.sum(
            jnp.where(cols_s == rows_s, mw_full, 0.0), axis=0, keepdims=True)
        m2_full = jax.lax.dot_general(mean, mean, _DN_T,
                                      preferred_element_type=f32)
        cols_n = jax.lax.broadcasted_iota(jnp.int32, (_N, _N), 1)
        rows_n = jax.lax.broadcasted_iota(jnp.int32, (_N, _N), 0)
        sm_s[0:1, _N * _S:_N * _S + _N] = jnp.sum(
            jnp.where(cols_n == rows_n, m2_full, 0.0), axis=0, keepdims=True)
        c_full = jax.lax.dot_general(
            db_ref[...], wcat[:, :_N * _H], _DN_N,
            preferred_element_type=f32)
        bias_eff = jnp.concatenate([
            bias_ref[n:n + 1, :] - jax.lax.dot_general(
                c_full[n:n + 1, n * _H:(n + 1) * _H], ld_ref[n], _DN_T,
                preferred_element_type=f32)
            for n in range(_N)], axis=0)
        gt_s[...] = jnp.concatenate(
            [ld_ref[n] for n in range(_N)] + [bias_eff.T],
            axis=1).astype(bf16)

    x_bf = x_ref[...].astype(bf16)
    x_lin = jax.lax.dot_general(
        x_bf, w_s[...], _DN_T,
        preferred_element_type=f32) + b_ref[...]
    aux = jax.lax.dot_general(
        x_bf, fw_s[...], _DN_N,
        preferred_element_type=f32) + sm_s[1:2, 0:_NK]
    u_all = aux[:, :_N * _H]
    proj = aux[:, _N * _H:_N * _H + _N * _S] - sm_s[0:1, 0:_N * _S]
    xm = aux[:, _N * _H + _N * _S:]

    r2 = jnp.sum(x_lin * x_lin, axis=1, keepdims=True)
    d2 = r2 - 2.0 * xm + sm_s[0:1, _N * _S:_N * _S + _N]
    q = proj * proj
    smat = (jax.lax.broadcasted_iota(jnp.int32, (_N * _S, _N), 0) // _S ==
            jax.lax.broadcasted_iota(jnp.int32, (_N * _S, _N), 1)).astype(f32)
    qsum = jax.lax.dot_general(
        q, smat, _DN_N, preferred_element_type=f32)
    score = slope_ref[...] * (qsum / d2 - center_ref[...])

    rowmax = jnp.max(score, axis=1, keepdims=True)
    tb = x_bf.shape[0]
    iota_n = jax.lax.broadcasted_iota(jnp.int32, (tb, _N), 1)
    idx = jnp.min(jnp.where(score == rowmax, iota_n, _N),
                  axis=1, keepdims=True)
    ss = jax.nn.sigmoid(rowmax)

    nh = _N * _H
    vals = jnp.concatenate(
        [u_all, jnp.ones((tb, _N), dtype=f32)], axis=1)
    lbl = jax.lax.broadcasted_iota(jnp.int32, (tb, nh + _N), 1)
    lbl = jnp.where(lbl < nh, lbl // _H, lbl - nh)
    masked = jnp.where(lbl == idx, vals, 0.0).astype(bf16)
    upd = jax.lax.dot_general(
        masked, gt_s[...], _DN_T,
        preferred_element_type=f32)

    out_ref[...] = x_lin + ss * (_ETA * upd - x_lin)


def kernel(x, W_lin, b_lin, select_weight, select_mean_diff, imp_slope,
           imp_center, lora_update, lora_degen, bias_p, debias_p):
    B, T, D = x.shape
    N, _, S = select_weight.shape
    H = lora_update.shape[2]
    x2 = x.reshape(B * T, D)
    b2 = b_lin.reshape(1, D)
    slope = imp_slope.reshape(1, N)
    center = imp_center.reshape(1, N)

    TB = 512
    grid = ((B * T) // TB,)
    const = lambda shape: pl.BlockSpec(shape, lambda i: tuple(0 for _ in shape))
    out = pl.pallas_call(
        _glo_kernel,
        grid=grid,
        in_specs=[
            pl.BlockSpec((TB, D), lambda i: (i, 0)),
            const((D, D)),
            const((1, D)),
            const((N, D, S)),
            const((N, D)),
            const((1, N)),
            const((1, N)),
            const((N, D, H)),
            const((N, D, H)),
            const((N, D)),
            const((N, D)),
        ],
        out_specs=pl.BlockSpec((TB, D), lambda i: (i, 0)),
        out_shape=jax.ShapeDtypeStruct((B * T, D), jnp.float32),
        scratch_shapes=[
            pltpu.VMEM((D, D), jnp.bfloat16),
            pltpu.VMEM((D, _NK), jnp.bfloat16),
            pltpu.VMEM((D, _N * (_H + 1)), jnp.bfloat16),
            pltpu.VMEM((8, 128), jnp.float32),
        ],
        compiler_params=pltpu.CompilerParams(
            dimension_semantics=("arbitrary",)),
    )(x2, W_lin, b2, select_weight, select_mean_diff, slope, center,
      lora_update, lora_degen, bias_p, debias_p)
    return out.reshape(B, T, D)
